# baseline (speedup 1.0000x reference)
import functools

import jax
import jax.numpy as jnp
from jax import lax
from jax.experimental import pallas as pl
from jax.experimental.pallas import tpu as pltpu
from jax.experimental.pallas import tpu_sc as plsc

N = 6_400_000
N_STRUCTURES = 16384
N_SPECIES_PAD = 128
NC, NS = 2, 16
NW = NC * NS
CHUNK = N // NW
BLK = 10000
NBLK = CHUNK // BLK
L = 16


def _sc_body(le_hbm, z_hbm, b_hbm, tab_hbm, out_hbm,
             tab_v, le0_v, le1_v, z0_v, z1_v, b0_v, b1_v,
             acc_v, sem0, sem1):
    c = lax.axis_index("c")
    s = lax.axis_index("s")
    wid = s * NC + c
    base = wid * CHUNK

    pltpu.sync_copy(tab_hbm, tab_v)

    zeros16 = jnp.zeros((L,), jnp.float32)

    def zero_body(i, carry):
        acc_v[pl.ds(i * L, L)] = zeros16
        return carry

    lax.fori_loop(0, N_STRUCTURES // L, zero_body, 0, unroll=8)

    bufs = ((le0_v, z0_v, b0_v, sem0), (le1_v, z1_v, b1_v, sem1))

    def start_fetch(g):
        le_b, z_b, b_b, sem = bufs[g % 2]
        off = base + g * BLK
        return (
            pltpu.async_copy(le_hbm.at[pl.ds(off, BLK)], le_b, sem),
            pltpu.async_copy(z_hbm.at[pl.ds(off, BLK)], z_b, sem),
            pltpu.async_copy(b_hbm.at[pl.ds(off, BLK)], b_b, sem),
        )

    def compute_block(g, carry):
        le_b, z_b, b_b, _ = bufs[g % 2]

        def energy(j):
            jl = j * L
            zz = z_b[pl.ds(jl, L)]
            w = plsc.load_gather(tab_v, [zz])
            sc = plsc.bitcast(jnp.left_shift(w, 16), jnp.float32)
            sh = plsc.bitcast(jnp.bitwise_and(w, jnp.int32(-65536)),
                              jnp.float32)
            return le_b[pl.ds(jl, L)] * sc + sh

        def vec_body(j, carry2):
            run_sum, prev_bb = carry2
            bb = b_b[pl.ds(j * L, L)]
            e = energy(j)
            chg = bb != prev_bb
            plsc.addupdate_scatter(acc_v, [prev_bb], run_sum, mask=chg)
            run_sum2 = jnp.where(chg, e, run_sum + e)
            return run_sum2, bb

        init = (energy(0), b_b[pl.ds(0, L)])
        run_sum, prev_bb = lax.fori_loop(
            1, BLK // L, vec_body, init, unroll=13)
        plsc.addupdate_scatter(acc_v, [prev_bb], run_sum)
        return carry

    def start_fetch_dyn(blk_idx, bufidx):
        le_b, z_b, b_b, sem = bufs[bufidx]
        off = jnp.minimum(base + blk_idx * BLK, N - BLK)
        pltpu.async_copy(le_hbm.at[pl.ds(off, BLK)], le_b, sem)
        pltpu.async_copy(z_hbm.at[pl.ds(off, BLK)], z_b, sem)
        pltpu.async_copy(b_hbm.at[pl.ds(off, BLK)], b_b, sem)

    def wait_buf(bufidx):
        le_b, z_b, b_b, sem = bufs[bufidx]
        pltpu.make_async_copy(le_hbm.at[pl.ds(0, BLK)], le_b, sem).wait()
        pltpu.make_async_copy(z_hbm.at[pl.ds(0, BLK)], z_b, sem).wait()
        pltpu.make_async_copy(b_hbm.at[pl.ds(0, BLK)], b_b, sem).wait()

    start_fetch(0)
    start_fetch(1)
    wait_buf(0)

    def pair_body(p, carry):
        carry = compute_block(0, carry)
        start_fetch_dyn(2 * p + 2, 0)
        wait_buf(1)
        carry = compute_block(1, carry)
        start_fetch_dyn(2 * p + 3, 1)
        wait_buf(0)
        return carry

    lax.fori_loop(0, NBLK // 2, pair_body, 0)
    wait_buf(1)

    pltpu.sync_copy(acc_v, out_hbm.at[wid])


@functools.partial(
    pl.kernel,
    out_type=jax.ShapeDtypeStruct((NW, N_STRUCTURES), jnp.float32),
    mesh=plsc.VectorSubcoreMesh(core_axis_name="c", subcore_axis_name="s"),
    scratch_types=[
        pltpu.VMEM((N_SPECIES_PAD,), jnp.int32),
        pltpu.VMEM((BLK,), jnp.float32),
        pltpu.VMEM((BLK,), jnp.float32),
        pltpu.VMEM((BLK,), jnp.int32),
        pltpu.VMEM((BLK,), jnp.int32),
        pltpu.VMEM((BLK,), jnp.int32),
        pltpu.VMEM((BLK,), jnp.int32),
        pltpu.VMEM((N_STRUCTURES,), jnp.float32),
        pltpu.SemaphoreType.DMA,
        pltpu.SemaphoreType.DMA,
    ],
    compiler_params=pltpu.CompilerParams(needs_layout_passes=False),
)
def _sc_partial_sums(*args):
    _sc_body(*args)


def _merge_body(parts_ref, out_ref):
    out_ref[...] = jnp.sum(parts_ref[...], axis=0)


def kernel(local_energies, Z, batch, scale, shift):
    sc16 = lax.bitcast_convert_type(
        scale.astype(jnp.bfloat16), jnp.uint16).astype(jnp.uint32)
    sh16 = lax.bitcast_convert_type(
        shift.astype(jnp.bfloat16), jnp.uint16).astype(jnp.uint32)
    tab = lax.bitcast_convert_type(
        jnp.left_shift(sh16, 16) | sc16, jnp.int32)
    tab_p = jnp.zeros((N_SPECIES_PAD,), jnp.int32).at[: tab.shape[0]].set(tab)
    parts = _sc_partial_sums(local_energies, Z, batch, tab_p)
    total = pl.pallas_call(
        _merge_body,
        out_shape=jax.ShapeDtypeStruct((N_STRUCTURES,), jnp.float32),
    )(parts)
    return total


# speedup vs baseline: 1.0684x; 1.0684x over previous
import functools

import jax
import jax.numpy as jnp
from jax import lax
from jax.experimental import pallas as pl
from jax.experimental.pallas import tpu as pltpu
from jax.experimental.pallas import tpu_sc as plsc

N = 6_400_000
N_STRUCTURES = 16384
N_SPECIES_PAD = 128
NC, NS = 2, 16
NW = NC * NS
CHUNK = N // NW
BLK = 10000
NBLK = CHUNK // BLK
L = 16


def _sc_body(le_hbm, z_hbm, b_hbm, tab_hbm, out_hbm,
             tab_v, le0_v, le1_v, z0_v, z1_v, b0_v, b1_v,
             acc_v, sem0, sem1):
    c = lax.axis_index("c")
    s = lax.axis_index("s")
    wid = s * NC + c
    base = wid * CHUNK

    pltpu.sync_copy(tab_hbm, tab_v)

    zeros16 = jnp.zeros((L,), jnp.float32)

    def zero_body(i, carry):
        acc_v[pl.ds(i * L, L)] = zeros16
        return carry

    lax.fori_loop(0, N_STRUCTURES // L, zero_body, 0, unroll=8)

    bufs = ((le0_v, z0_v, b0_v, sem0), (le1_v, z1_v, b1_v, sem1))

    def start_fetch(g):
        le_b, z_b, b_b, sem = bufs[g % 2]
        off = base + g * BLK
        return (
            pltpu.async_copy(le_hbm.at[pl.ds(off, BLK)], le_b, sem),
            pltpu.async_copy(z_hbm.at[pl.ds(off, BLK)], z_b, sem),
            pltpu.async_copy(b_hbm.at[pl.ds(off, BLK)], b_b, sem),
        )

    def compute_block(g, carry):
        le_b, z_b, b_b, _ = bufs[g % 2]

        def energy(j):
            jl = j * L
            zz = z_b[pl.ds(jl, L)]
            w = plsc.load_gather(tab_v, [zz])
            sc = plsc.bitcast(jnp.left_shift(w, 16), jnp.float32)
            sh = plsc.bitcast(jnp.bitwise_and(w, jnp.int32(-65536)),
                              jnp.float32)
            return le_b[pl.ds(jl, L)] * sc + sh

        def vec_body(j, carry2):
            run_sum, prev_bb = carry2
            bb = b_b[pl.ds(j * L, L)]
            e = energy(j)
            chg = bb != prev_bb
            plsc.addupdate_scatter(acc_v, [prev_bb], run_sum, mask=chg)
            run_sum2 = jnp.where(chg, e, run_sum + e)
            return run_sum2, bb

        init = (energy(0), b_b[pl.ds(0, L)])
        run_sum, prev_bb = lax.fori_loop(
            1, BLK // L, vec_body, init, unroll=5)
        plsc.addupdate_scatter(acc_v, [prev_bb], run_sum)
        return carry

    def start_fetch_dyn(blk_idx, bufidx):
        le_b, z_b, b_b, sem = bufs[bufidx]
        off = jnp.minimum(base + blk_idx * BLK, N - BLK)
        pltpu.async_copy(le_hbm.at[pl.ds(off, BLK)], le_b, sem)
        pltpu.async_copy(z_hbm.at[pl.ds(off, BLK)], z_b, sem)
        pltpu.async_copy(b_hbm.at[pl.ds(off, BLK)], b_b, sem)

    def wait_buf(bufidx):
        le_b, z_b, b_b, sem = bufs[bufidx]
        pltpu.make_async_copy(le_hbm.at[pl.ds(0, BLK)], le_b, sem).wait()
        pltpu.make_async_copy(z_hbm.at[pl.ds(0, BLK)], z_b, sem).wait()
        pltpu.make_async_copy(b_hbm.at[pl.ds(0, BLK)], b_b, sem).wait()

    start_fetch(0)
    start_fetch(1)
    wait_buf(0)

    def pair_body(p, carry):
        carry = compute_block(0, carry)
        start_fetch_dyn(2 * p + 2, 0)
        wait_buf(1)
        carry = compute_block(1, carry)
        start_fetch_dyn(2 * p + 3, 1)
        wait_buf(0)
        return carry

    lax.fori_loop(0, NBLK // 2, pair_body, 0)
    wait_buf(1)

    pltpu.sync_copy(acc_v, out_hbm.at[wid])


@functools.partial(
    pl.kernel,
    out_type=jax.ShapeDtypeStruct((NW, N_STRUCTURES), jnp.float32),
    mesh=plsc.VectorSubcoreMesh(core_axis_name="c", subcore_axis_name="s"),
    scratch_types=[
        pltpu.VMEM((N_SPECIES_PAD,), jnp.int32),
        pltpu.VMEM((BLK,), jnp.float32),
        pltpu.VMEM((BLK,), jnp.float32),
        pltpu.VMEM((BLK,), jnp.int32),
        pltpu.VMEM((BLK,), jnp.int32),
        pltpu.VMEM((BLK,), jnp.int32),
        pltpu.VMEM((BLK,), jnp.int32),
        pltpu.VMEM((N_STRUCTURES,), jnp.float32),
        pltpu.SemaphoreType.DMA,
        pltpu.SemaphoreType.DMA,
    ],
    compiler_params=pltpu.CompilerParams(needs_layout_passes=False),
)
def _sc_partial_sums(*args):
    _sc_body(*args)


def _merge_body(parts_ref, out_ref):
    out_ref[...] = jnp.sum(parts_ref[...], axis=0)


def kernel(local_energies, Z, batch, scale, shift):
    sc16 = lax.bitcast_convert_type(
        scale.astype(jnp.bfloat16), jnp.uint16).astype(jnp.uint32)
    sh16 = lax.bitcast_convert_type(
        shift.astype(jnp.bfloat16), jnp.uint16).astype(jnp.uint32)
    tab = lax.bitcast_convert_type(
        jnp.left_shift(sh16, 16) | sc16, jnp.int32)
    tab_p = jnp.zeros((N_SPECIES_PAD,), jnp.int32).at[: tab.shape[0]].set(tab)
    parts = _sc_partial_sums(local_energies, Z, batch, tab_p)
    total = pl.pallas_call(
        _merge_body,
        out_shape=jax.ShapeDtypeStruct((N_STRUCTURES,), jnp.float32),
    )(parts)
    return total
